# baseline (speedup 1.0000x reference)
import functools

import jax
import jax.numpy as jnp
from jax import lax
from jax.experimental import pallas as pl
from jax.experimental.pallas import tpu as pltpu
from jax.experimental.pallas import tpu_sc as plsc

B = 4096
S = 200
E = 64
L = 16
NC = 2
NS = 16
NW = NC * NS
BW = B // NW
SCALE = 8.0


def _sc_embed(src_t, token_table, pos_table):
    mesh = plsc.VectorSubcoreMesh(core_axis_name="c", subcore_axis_name="s")

    @functools.partial(
        pl.kernel,
        mesh=mesh,
        compiler_params=pltpu.CompilerParams(use_tc_tiling_on_sc=False),
        out_type=jax.ShapeDtypeStruct((B, S * E), jnp.float32),
        scratch_types=[
            pltpu.VMEM((S, BW), jnp.int32),
            pltpu.VMEM((S, E), jnp.float32),
            pltpu.VMEM((2 * BW, E), jnp.float32),
            pltpu.VMEM((BW, 2 * E), jnp.float32),
            pltpu.SemaphoreType.DMA,
        ],
    )
    def body(src_hbm, tab_hbm, pos_hbm, out_hbm, idx_v, pos_v, gbuf, obuf, sem):
        w = lax.axis_index("s") * NC + lax.axis_index("c")
        b0 = w * BW
        pltpu.sync_copy(src_hbm.at[:, pl.ds(b0, BW)], idx_v)
        pltpu.sync_copy(pos_hbm, pos_v)

        @pl.loop(0, S // 2)
        def _per_pair(j):
            s = j * 2
            cp0 = pltpu.async_copy(
                tab_hbm.at[idx_v.at[s]], gbuf.at[pl.ds(0, BW)], sem)
            cp1 = pltpu.async_copy(
                tab_hbm.at[idx_v.at[s + 1]], gbuf.at[pl.ds(BW, BW)], sem)
            cp0.wait()
            cp1.wait()
            for half in range(2):
                p0 = pos_v[s + half, pl.ds(0 * L, L)]
                p1 = pos_v[s + half, pl.ds(1 * L, L)]
                p2 = pos_v[s + half, pl.ds(2 * L, L)]
                p3 = pos_v[s + half, pl.ds(3 * L, L)]
                g0 = half * BW
                o0 = half * E

                @pl.loop(0, BW)
                def _per_row(r):
                    obuf[r, pl.ds(o0 + 0 * L, L)] = (
                        gbuf[g0 + r, pl.ds(0 * L, L)] * SCALE + p0)
                    obuf[r, pl.ds(o0 + 1 * L, L)] = (
                        gbuf[g0 + r, pl.ds(1 * L, L)] * SCALE + p1)
                    obuf[r, pl.ds(o0 + 2 * L, L)] = (
                        gbuf[g0 + r, pl.ds(2 * L, L)] * SCALE + p2)
                    obuf[r, pl.ds(o0 + 3 * L, L)] = (
                        gbuf[g0 + r, pl.ds(3 * L, L)] * SCALE + p3)

            pltpu.sync_copy(
                obuf, out_hbm.at[pl.ds(b0, BW), pl.ds(j * 2 * E, 2 * E)])

    return body(src_t, token_table, pos_table)


def kernel(src, tgt, token_table, pos_table):
    del tgt
    src_t = src.T
    out = _sc_embed(src_t, token_table, pos_table)
    return out.reshape(B, S, E)


# speedup vs baseline: 1.1398x; 1.1398x over previous
import functools

import jax
import jax.numpy as jnp
from jax import lax
from jax.experimental import pallas as pl
from jax.experimental.pallas import tpu as pltpu
from jax.experimental.pallas import tpu_sc as plsc

B = 4096
S = 200
E = 64
L = 16
NC = 2
NS = 16
NW = NC * NS
BW = B // NW
NPAIR = S // 2
SCALE = 8.0


def _sc_embed(src_t, token_table, pos_table):
    mesh = plsc.VectorSubcoreMesh(core_axis_name="c", subcore_axis_name="s")

    @functools.partial(
        pl.kernel,
        mesh=mesh,
        compiler_params=pltpu.CompilerParams(use_tc_tiling_on_sc=False),
        out_type=jax.ShapeDtypeStruct((B, S * E), jnp.float32),
        scratch_types=[
            pltpu.VMEM((S, BW), jnp.int32),
            pltpu.VMEM((S, E), jnp.float32),
            pltpu.VMEM((2, 2 * BW, E), jnp.float32),
            pltpu.VMEM((2, BW, 2 * E), jnp.float32),
            pltpu.SemaphoreType.DMA,
            pltpu.SemaphoreType.DMA,
            pltpu.SemaphoreType.DMA,
            pltpu.SemaphoreType.DMA,
        ],
    )
    def body(src_hbm, tab_hbm, pos_hbm, out_hbm, idx_v, pos_v, gbuf, obuf,
             gsem0, gsem1, osem0, osem1):
        w = lax.axis_index("s") * NC + lax.axis_index("c")
        b0 = w * BW
        pltpu.sync_copy(src_hbm.at[:, pl.ds(b0, BW)], idx_v)
        pltpu.sync_copy(pos_hbm, pos_v)
        gsems = (gsem0, gsem1)
        osems = (osem0, osem1)

        def start_gather(p, slot):
            s = p * 2
            pltpu.async_copy(
                tab_hbm.at[idx_v.at[s]], gbuf.at[slot, pl.ds(0, BW)],
                gsems[slot])
            pltpu.async_copy(
                tab_hbm.at[idx_v.at[s + 1]], gbuf.at[slot, pl.ds(BW, BW)],
                gsems[slot])

        def wait_gather(slot):
            pltpu.make_async_copy(
                tab_hbm.at[pl.ds(0, 2 * BW)], gbuf.at[slot],
                gsems[slot]).wait()

        def out_window(p):
            return out_hbm.at[pl.ds(b0, BW), pl.ds(p * 2 * E, 2 * E)]

        start_gather(0, 0)

        @pl.loop(0, NPAIR, step=2)
        def _pairs(j):
            for bslot in range(2):
                p = j + bslot
                s = p * 2

                @pl.when(p + 1 < NPAIR)
                def _():
                    start_gather(p + 1, (bslot + 1) % 2)

                wait_gather(bslot)

                @pl.when(p >= 2)
                def _():
                    pltpu.make_async_copy(
                        obuf.at[bslot], out_window(p - 2),
                        osems[bslot]).wait()

                for half in range(2):
                    g0 = half * BW
                    o0 = half * E
                    p0 = pos_v[s + half, pl.ds(0 * L, L)]
                    p1 = pos_v[s + half, pl.ds(1 * L, L)]
                    p2 = pos_v[s + half, pl.ds(2 * L, L)]
                    p3 = pos_v[s + half, pl.ds(3 * L, L)]
                    pv = (p0, p1, p2, p3)

                    @pl.loop(0, BW, step=4)
                    def _rows(r, _g0=g0, _o0=o0, _pv=pv, _slot=bslot):
                        for dr in range(4):
                            for c in range(E // L):
                                obuf[_slot, r + dr,
                                     pl.ds(_o0 + c * L, L)] = (
                                    gbuf[_slot, _g0 + r + dr,
                                         pl.ds(c * L, L)] * SCALE + _pv[c])

                pltpu.async_copy(obuf.at[bslot], out_window(p), osems[bslot])

        pltpu.make_async_copy(
            obuf.at[0], out_window(NPAIR - 2), osems[0]).wait()
        pltpu.make_async_copy(
            obuf.at[1], out_window(NPAIR - 1), osems[1]).wait()

    return body(src_t, token_table, pos_table)


def kernel(src, tgt, token_table, pos_table):
    del tgt
    src_t = src.T
    out = _sc_embed(src_t, token_table, pos_table)
    return out.reshape(B, S, E)


# speedup vs baseline: 1.3952x; 1.2241x over previous
import functools

import jax
import jax.numpy as jnp
from jax import lax
from jax.experimental import pallas as pl
from jax.experimental.pallas import tpu as pltpu
from jax.experimental.pallas import tpu_sc as plsc

B = 4096
S = 200
E = 64
L = 16
NC = 2
NS = 16
NW = NC * NS
SW = B // NW
NBUF = 4
SCALE = 8.0


def _sc_embed(src, token_table, pos_table):
    mesh = plsc.VectorSubcoreMesh(core_axis_name="c", subcore_axis_name="s")

    @functools.partial(
        pl.kernel,
        mesh=mesh,
        compiler_params=pltpu.CompilerParams(use_tc_tiling_on_sc=False),
        out_type=jax.ShapeDtypeStruct((B, S, E), jnp.float32),
        scratch_types=[
            pltpu.VMEM((SW, S), jnp.int32),
            pltpu.VMEM((S, E), jnp.float32),
            pltpu.VMEM((NBUF, S, E), jnp.float32),
            [pltpu.SemaphoreType.DMA] * NBUF,
            [pltpu.SemaphoreType.DMA] * NBUF,
        ],
    )
    def body(src_hbm, tab_hbm, pos_hbm, out_hbm, idx_v, pos_v, ring,
             gsems, osems):
        w = lax.axis_index("s") * NC + lax.axis_index("c")
        q0 = w * SW
        pltpu.sync_copy(src_hbm.at[pl.ds(q0, SW)], idx_v)
        pltpu.sync_copy(pos_hbm, pos_v)

        def start_gather(seq, slot):
            pltpu.async_copy(
                tab_hbm.at[idx_v.at[seq, pl.ds(0, 128)]],
                ring.at[slot, pl.ds(0, 128)], gsems[slot])
            pltpu.async_copy(
                tab_hbm.at[idx_v.at[seq, pl.ds(128, S - 128)]],
                ring.at[slot, pl.ds(128, S - 128)], gsems[slot])

        def wait_gather(slot):
            pltpu.make_async_copy(
                tab_hbm.at[pl.ds(0, S)], ring.at[slot], gsems[slot]).wait()

        def wait_write(slot):
            pltpu.make_async_copy(
                ring.at[slot], out_hbm.at[q0], osems[slot]).wait()

        start_gather(0, 0)
        start_gather(1, 1)

        @pl.loop(0, SW, step=NBUF)
        def _seqs(j):
            for k in range(NBUF):
                p = j + k
                wait_gather(k)

                @pl.loop(0, S, step=2)
                def _rows(r, _k=k):
                    for dr in range(2):
                        for c in range(E // L):
                            ring[_k, r + dr, pl.ds(c * L, L)] = (
                                ring[_k, r + dr, pl.ds(c * L, L)] * SCALE
                                + pos_v[r + dr, pl.ds(c * L, L)])

                pltpu.async_copy(ring.at[k], out_hbm.at[q0 + p], osems[k])

                nxt = p + 2
                nslot = (k + 2) % NBUF

                @pl.when(nxt < SW)
                def _():
                    @pl.when(p >= 2)
                    def _():
                        wait_write(nslot)

                    start_gather(nxt, nslot)

        for k in range(NBUF):
            wait_write(k)

    return body(src, token_table, pos_table)


def kernel(src, tgt, token_table, pos_table):
    del tgt
    return _sc_embed(src, token_table, pos_table)
